# baseline (speedup 1.0000x reference)
import functools

import jax
import jax.numpy as jnp
from jax import lax
from jax.experimental import pallas as pl
from jax.experimental.pallas import tpu as pltpu
from jax.experimental.pallas import tpu_sc as plsc

D_EMB = 128
D_HID = 768

CHUNK = 128
NBUF = 5
SLICE_UNITS = (1, 2, 3, 4)
BLK = 4096


def _sc_gather(idx3, table, n_workers, n_chunks):
    tokens = n_workers * n_chunks * CHUNK
    per_w = n_chunks * CHUNK
    n_groups = n_chunks // NBUF
    d_half = D_EMB // 2
    info = plsc.get_sparse_core_info()
    nc = info.num_cores

    mesh = plsc.VectorSubcoreMesh(core_axis_name="c", subcore_axis_name="s")

    scratch = [pltpu.VMEM((n_chunks, CHUNK), jnp.int32)]
    scratch += [pltpu.VMEM((CHUNK, d_half), jnp.int32) for _ in range(NBUF)]
    scratch += [pltpu.SemaphoreType.DMA for _ in range(NBUF)]

    @functools.partial(
        pl.kernel,
        mesh=mesh,
        out_type=jax.ShapeDtypeStruct((tokens, d_half), jnp.int32),
        scratch_types=scratch,
        compiler_params=pltpu.CompilerParams(use_tc_tiling_on_sc=False),
    )
    def gather_k(idx_hbm, table_hbm, out_hbm, idx_v, *bufs_and_sems):
        bufs = bufs_and_sems[:NBUF]
        sems = bufs_and_sems[NBUF:]
        wid = lax.axis_index("s") * nc + lax.axis_index("c")
        base_row = wid * per_w
        pltpu.sync_copy(idx_hbm.at[wid], idx_v)

        def body(g, carry):
            base_c = g * NBUF
            copies = []
            for bi in range(NBUF):
                cp = pltpu.make_async_copy(
                    table_hbm.at[idx_v.at[base_c + bi]], bufs[bi], sems[bi])
                cp.start()
                copies.append(cp)
            for bi in range(NBUF):
                copies[bi].wait()
                row0 = base_row + (base_c + bi) * CHUNK
                pltpu.sync_copy(bufs[bi], out_hbm.at[pl.ds(row0, CHUNK)])
            return carry

        lax.fori_loop(0, n_groups, body, 0)

    return gather_k(idx3, table)


def _tc_project_slice(emb_s, W, b2, buf, s, tokens):
    slice_tokens = emb_s.shape[0]
    g = slice_tokens // BLK
    blk0 = s
    d_half = D_EMB // 2

    def mm_body(emb_ref, we_ref, wo_ref, b_ref, *rest):
        out_ref = rest[-1]
        x = emb_ref[...]
        e_even = lax.bitcast_convert_type(x << 16, jnp.float32)
        e_odd = lax.bitcast_convert_type(
            x & jnp.int32(-65536), jnp.float32)
        acc = jnp.dot(e_even, we_ref[...], preferred_element_type=jnp.float32)
        acc += jnp.dot(e_odd, wo_ref[...], preferred_element_type=jnp.float32)
        out_ref[...] = acc + b_ref[...]

    in_specs = [
        pl.BlockSpec((BLK, d_half), lambda i: (i, 0)),
        pl.BlockSpec((d_half, D_HID), lambda i: (0, 0)),
        pl.BlockSpec((d_half, D_HID), lambda i: (0, 0)),
        pl.BlockSpec((1, D_HID), lambda i: (0, 0)),
    ]
    args = [emb_s] + list(W) + [b2]
    aliases = {}
    if buf is not None:
        in_specs.append(pl.BlockSpec(memory_space=pl.ANY))
        args.append(buf)
        aliases = {4: 0}

    return pl.pallas_call(
        mm_body,
        grid=(g,),
        in_specs=in_specs,
        out_specs=pl.BlockSpec((BLK, D_HID), lambda i: (blk0 + i, 0)),
        out_shape=jax.ShapeDtypeStruct((tokens, D_HID), jnp.float32),
        input_output_aliases=aliases,
    )(*args)


def _tc_mask(idx2):
    rows = idx2.shape[0]

    def mask_body(idx_ref, mask_ref):
        mask_ref[...] = idx_ref[...] != 0

    return pl.pallas_call(
        mask_body,
        grid=(8,),
        in_specs=[pl.BlockSpec((rows // 8, D_EMB), lambda i: (i, 0))],
        out_specs=pl.BlockSpec((rows // 8, D_EMB), lambda i: (i, 0)),
        out_shape=jax.ShapeDtypeStruct((rows, D_EMB), jnp.bool_),
    )(idx2)


def kernel(idxs, table, W, b):
    B, L = idxs.shape
    tokens = B * L
    info = plsc.get_sparse_core_info()
    n_workers = info.num_cores * info.num_subcores
    unit = NBUF * n_workers * CHUNK
    assert tokens == sum(SLICE_UNITS) * unit

    idx_flat = idxs.astype(jnp.int32).reshape(-1)
    b2 = b.reshape(1, D_HID)

    v = table.shape[0]
    table_i32 = lax.bitcast_convert_type(
        table.astype(jnp.bfloat16).reshape(v, D_EMB // 2, 2), jnp.int32)
    Wsplit = (W[0::2, :], W[1::2, :])

    embs = []
    t0 = 0
    for u in SLICE_UNITS:
        st = u * unit
        idx3 = lax.slice(idx_flat, (t0,), (t0 + st,)).reshape(
            n_workers, u * NBUF, CHUNK)
        embs.append(_sc_gather(idx3, table_i32, n_workers, u * NBUF))
        t0 += st

    buf = None
    t0 = 0
    for u, emb_s in zip(SLICE_UNITS, embs):
        buf = _tc_project_slice(emb_s, Wsplit, b2, buf, t0 // BLK, tokens)
        t0 += u * unit

    mask2 = _tc_mask(idx_flat.reshape(tokens // D_EMB, D_EMB))
    return buf.reshape(B, L, D_HID), mask2.reshape(B, L)


# speedup vs baseline: 2.9605x; 2.9605x over previous
import functools

import jax
import jax.numpy as jnp
from jax import lax
from jax.experimental import pallas as pl
from jax.experimental.pallas import tpu as pltpu
from jax.experimental.pallas import tpu_sc as plsc

D_EMB = 128
D_HID = 768

CHUNK = 128
NBUF = 5
SLICE_UNITS = (1, 2, 3, 4)
BLK = 4096


def _sc_gather(idx3, table, n_workers, n_chunks):
    tokens = n_workers * n_chunks * CHUNK
    per_w = n_chunks * CHUNK
    n_groups = n_chunks // NBUF
    info = plsc.get_sparse_core_info()
    nc = info.num_cores

    mesh = plsc.VectorSubcoreMesh(core_axis_name="c", subcore_axis_name="s")

    scratch = [pltpu.VMEM((n_chunks, CHUNK), jnp.int32)]
    scratch += [pltpu.VMEM((CHUNK, D_EMB), jnp.float32) for _ in range(NBUF)]
    scratch += [pltpu.VMEM((CHUNK // 2, D_EMB), jnp.int32)]
    scratch += [pltpu.SemaphoreType.DMA for _ in range(NBUF)]

    @functools.partial(
        pl.kernel,
        mesh=mesh,
        out_type=jax.ShapeDtypeStruct((tokens // 2, D_EMB), jnp.int32),
        scratch_types=scratch,
    )
    def gather_k(idx_hbm, table_hbm, out_hbm, idx_v, *bufs_and_sems):
        bufs = bufs_and_sems[:NBUF]
        packed = bufs_and_sems[NBUF]
        sems = bufs_and_sems[NBUF + 1:]
        wid = lax.axis_index("s") * nc + lax.axis_index("c")
        base_row = wid * per_w // 2
        pltpu.sync_copy(idx_hbm.at[wid], idx_v)
        hmask = jnp.full((16,), -65536, jnp.int32)

        def body(g, carry):
            base_c = g * NBUF
            copies = []
            for bi in range(NBUF):
                cp = pltpu.make_async_copy(
                    table_hbm.at[idx_v.at[base_c + bi]], bufs[bi], sems[bi])
                cp.start()
                copies.append(cp)
            for bi in range(NBUF):
                copies[bi].wait()

                def pack_row(j, c, buf=bufs[bi]):
                    a = lax.bitcast_convert_type(
                        buf[j, pl.ds(c, 16)], jnp.int32)
                    bv = lax.bitcast_convert_type(
                        buf[CHUNK // 2 + j, pl.ds(c, 16)], jnp.int32)
                    packed[j, pl.ds(c, 16)] = (
                        ((a >> 16) & ~hmask) | (bv & hmask))

                def pack_body(j, carry2):
                    for gc in range(D_EMB // 16):
                        pack_row(j, gc * 16)
                    return carry2

                lax.fori_loop(0, CHUNK // 2, pack_body, 0)
                row0 = pl.multiple_of(
                    base_row + (base_c + bi) * (CHUNK // 2), CHUNK // 2)
                pltpu.sync_copy(packed, out_hbm.at[pl.ds(row0, CHUNK // 2)])
            return carry

        lax.fori_loop(0, n_groups, body, 0)

    return gather_k(idx3, table)


def _tc_project_slice(emb_s, W, b2, buf, s, tokens):
    slice_rows = emb_s.shape[0]
    g = slice_rows // (BLK // 2)
    blk0 = s

    def mm_body(emb_ref, w_ref, b_ref, *rest):
        out_ref = rest[-1]
        x = emb_ref[...]
        e_even = lax.bitcast_convert_type(x << 16, jnp.float32)
        e_odd = lax.bitcast_convert_type(
            x & jnp.int32(-65536), jnp.float32)
        w = w_ref[...]
        out_ref[0] = jnp.dot(
            e_even, w, preferred_element_type=jnp.float32) + b_ref[...]
        out_ref[1] = jnp.dot(
            e_odd, w, preferred_element_type=jnp.float32) + b_ref[...]

    in_specs = [
        pl.BlockSpec((BLK // 2, D_EMB), lambda i: (i, 0)),
        pl.BlockSpec((D_EMB, D_HID), lambda i: (0, 0)),
        pl.BlockSpec((1, D_HID), lambda i: (0, 0)),
    ]
    args = [emb_s, W, b2]
    aliases = {}
    if buf is not None:
        in_specs.append(pl.BlockSpec(memory_space=pl.ANY))
        args.append(buf)
        aliases = {3: 0}

    return pl.pallas_call(
        mm_body,
        grid=(g,),
        in_specs=in_specs,
        out_specs=pl.BlockSpec((2, BLK // 2, D_HID),
                               lambda i: (0, blk0 + i, 0)),
        out_shape=jax.ShapeDtypeStruct((2, tokens // 2, D_HID), jnp.float32),
        input_output_aliases=aliases,
    )(*args)


def _tc_mask(idx2):
    rows = idx2.shape[0]

    def mask_body(idx_ref, mask_ref):
        mask_ref[...] = idx_ref[...] != 0

    return pl.pallas_call(
        mask_body,
        grid=(8,),
        in_specs=[pl.BlockSpec((rows // 8, D_EMB), lambda i: (i, 0))],
        out_specs=pl.BlockSpec((rows // 8, D_EMB), lambda i: (i, 0)),
        out_shape=jax.ShapeDtypeStruct((rows, D_EMB), jnp.bool_),
    )(idx2)


def kernel(idxs, table, W, b):
    B, L = idxs.shape
    tokens = B * L
    info = plsc.get_sparse_core_info()
    n_workers = info.num_cores * info.num_subcores
    unit = NBUF * n_workers * CHUNK
    assert tokens == sum(SLICE_UNITS) * unit

    idx_flat = idxs.astype(jnp.int32).reshape(-1)
    b2 = b.reshape(1, D_HID)

    half_groups = tokens // 128
    inter = jnp.transpose(
        idx_flat.reshape(2, half_groups, 64), (1, 0, 2)).reshape(
            half_groups, CHUNK)

    embs = []
    g0 = 0
    for u in SLICE_UNITS:
        ng = u * unit // CHUNK
        idx3 = lax.slice(inter, (g0, 0), (g0 + ng, CHUNK)).reshape(
            n_workers, u * NBUF, CHUNK)
        embs.append(_sc_gather(idx3, table, n_workers, u * NBUF))
        g0 += ng

    buf = None
    t0 = 0
    for u, emb_s in zip(SLICE_UNITS, embs):
        buf = _tc_project_slice(emb_s, W, b2, buf, t0 // BLK, tokens)
        t0 += u * unit

    mask2 = _tc_mask(idx_flat.reshape(tokens // D_EMB, D_EMB))
    return buf.reshape(B, L, D_HID), mask2.reshape(B, L)
